# baseline (speedup 1.0000x reference)
import jax
import jax.numpy as jnp
from jax import lax
from jax.experimental import pallas as pl
from jax.experimental.pallas import tpu as pltpu
from jax.experimental.pallas import tpu_sc as plsc

_B = 16384
_D = 16
_NC = 2
_NS = 16
_NW = _NC * _NS
_BW = _B // _NW
_G = _BW // 16


def _pairwise_body(u_idx_hbm, p_idx_hbm, n_idx_hbm, user_hbm, item_hbm,
                   out_hbm, uidx_v, pidx_v, nidx_v, u_v, i_v, j_v, out_v,
                   sem_u, sem_i, sem_j):
    wid = lax.axis_index("s") * _NC + lax.axis_index("c")
    base = wid * _BW
    pltpu.sync_copy(u_idx_hbm.at[pl.ds(base, _BW)], uidx_v)
    pltpu.sync_copy(p_idx_hbm.at[pl.ds(base, _BW)], pidx_v)
    pltpu.sync_copy(n_idx_hbm.at[pl.ds(base, _BW)], nidx_v)
    cu = pltpu.async_copy(user_hbm.at[uidx_v], u_v, sem_u)
    ci = pltpu.async_copy(item_hbm.at[pidx_v], i_v, sem_i)
    cj = pltpu.async_copy(item_hbm.at[nidx_v], j_v, sem_j)
    cu.wait()
    ci.wait()
    cj.wait()

    lane = lax.iota(jnp.int32, 16)

    def group(g, carry):
        r0 = g * 16
        rows = r0 + lane
        acc = jnp.zeros((16,), jnp.float32)
        for d in range(16):
            dv = jnp.full((16,), d, jnp.int32)
            gu = plsc.load_gather(u_v, [rows, dv])
            gi = plsc.load_gather(i_v, [rows, dv])
            gj = plsc.load_gather(j_v, [rows, dv])
            acc = acc + gu * (gi - gj)
        out_v[pl.ds(r0, 16)] = acc
        return carry

    lax.fori_loop(0, _G, group, 0)
    pltpu.sync_copy(out_v, out_hbm.at[pl.ds(base, _BW)])


def kernel(user_input, pos_item_input, neg_item_input, user_emb, item_emb):
    u_idx = user_input.reshape(-1).astype(jnp.int32)
    p_idx = pos_item_input.reshape(-1).astype(jnp.int32)
    n_idx = neg_item_input.reshape(-1).astype(jnp.int32)
    mesh = plsc.VectorSubcoreMesh(core_axis_name="c", subcore_axis_name="s")
    out = pl.kernel(
        _pairwise_body,
        out_type=jax.ShapeDtypeStruct((_B,), jnp.float32),
        mesh=mesh,
        compiler_params=pltpu.CompilerParams(
            needs_layout_passes=False, use_tc_tiling_on_sc=False),
        scratch_types=[
            pltpu.VMEM((_BW,), jnp.int32),
            pltpu.VMEM((_BW,), jnp.int32),
            pltpu.VMEM((_BW,), jnp.int32),
            pltpu.VMEM((_BW, _D), jnp.float32),
            pltpu.VMEM((_BW, _D), jnp.float32),
            pltpu.VMEM((_BW, _D), jnp.float32),
            pltpu.VMEM((_BW,), jnp.float32),
            pltpu.SemaphoreType.DMA,
            pltpu.SemaphoreType.DMA,
            pltpu.SemaphoreType.DMA,
        ],
    )(u_idx, p_idx, n_idx, user_emb, item_emb)
    dd = out.reshape(_B, 1)
    return (dd, dd)
